# Initial kernel scaffold; baseline (speedup 1.0000x reference)
import jax
import jax.numpy as jnp
from jax.experimental import pallas as pl


def kernel(deriv_types, embedding_weight):
    raise NotImplementedError("write your pallas kernel here")


# speedup vs baseline: 1.0396x; 1.0396x over previous
import functools

import jax
import jax.numpy as jnp
from jax import lax
from jax.experimental import pallas as pl
from jax.experimental.pallas import tpu as pltpu
from jax.experimental.pallas import tpu_sc as plsc

NUM_TYPES = 22
HIDDEN_DIM = 256
N_TOKENS = 16384

_NC = 2
_NS = 16
_NW = _NC * _NS
_ROWS_PER_W = N_TOKENS // _NW
_CHUNK = 128
_NCHUNKS = _ROWS_PER_W // _CHUNK


def _make_sc_gather():
  mesh = plsc.VectorSubcoreMesh(core_axis_name="c", subcore_axis_name="s")

  @functools.partial(
      pl.kernel,
      mesh=mesh,
      out_type=jax.ShapeDtypeStruct((N_TOKENS, HIDDEN_DIM), jnp.float32),
      scratch_types=[
          pltpu.VMEM((_NCHUNKS, _CHUNK), jnp.int32),
          pltpu.VMEM((_CHUNK, HIDDEN_DIM), jnp.float32),
          pltpu.SemaphoreType.DMA,
      ],
  )
  def k(idx_hbm, table_hbm, out_hbm, idx_v, rows_v, sem):
    wid = lax.axis_index("s") * _NC + lax.axis_index("c")
    base = wid * _ROWS_PER_W
    pltpu.sync_copy(idx_hbm.at[wid], idx_v)
    for j in range(_NCHUNKS):
      pltpu.async_copy(table_hbm.at[idx_v.at[j]], rows_v, sem).wait()
      pltpu.sync_copy(rows_v, out_hbm.at[pl.ds(base + j * _CHUNK, _CHUNK)])

  return k


_sc_gather = _make_sc_gather()


def kernel(deriv_types, embedding_weight):
  idx = deriv_types.astype(jnp.int32).reshape(_NW, _NCHUNKS, _CHUNK)
  return _sc_gather(idx, embedding_weight)


# speedup vs baseline: 1.0983x; 1.0565x over previous
import functools

import jax
import jax.numpy as jnp
from jax import lax
from jax.experimental import pallas as pl
from jax.experimental.pallas import tpu as pltpu
from jax.experimental.pallas import tpu_sc as plsc

NUM_TYPES = 22
HIDDEN_DIM = 256
N_TOKENS = 16384

_NC = 2
_NS = 16
_NW = _NC * _NS
_ROWS_PER_W = N_TOKENS // _NW
_CHUNK = 128
_NCHUNKS = _ROWS_PER_W // _CHUNK
_NBUF = 3


def _make_sc_gather():
  mesh = plsc.VectorSubcoreMesh(core_axis_name="c", subcore_axis_name="s")

  @functools.partial(
      pl.kernel,
      mesh=mesh,
      out_type=jax.ShapeDtypeStruct((N_TOKENS, HIDDEN_DIM), jnp.float32),
      scratch_types=(
          [pltpu.VMEM((_NCHUNKS, _CHUNK), jnp.int32)]
          + [pltpu.VMEM((_CHUNK, HIDDEN_DIM), jnp.float32)] * _NBUF
          + [pltpu.SemaphoreType.DMA] * (2 * _NBUF)
      ),
  )
  def k(idx_hbm, table_hbm, out_hbm, idx_v, *rest):
    bufs = rest[:_NBUF]
    gsem = rest[_NBUF:2 * _NBUF]
    ssem = rest[2 * _NBUF:]
    wid = lax.axis_index("s") * _NC + lax.axis_index("c")
    base = wid * _ROWS_PER_W
    pltpu.sync_copy(idx_hbm.at[wid], idx_v)

    def gather(j):
      b = j % _NBUF
      return pltpu.async_copy(table_hbm.at[idx_v.at[j]], bufs[b], gsem[b])

    def store(j):
      b = j % _NBUF
      return pltpu.async_copy(
          bufs[b], out_hbm.at[pl.ds(base + j * _CHUNK, _CHUNK)], ssem[b])

    gathers = [None] * _NCHUNKS
    stores = [None] * _NCHUNKS
    for j in range(min(_NBUF, _NCHUNKS)):
      gathers[j] = gather(j)
    for j in range(_NCHUNKS):
      gathers[j].wait()
      stores[j] = store(j)
      if j + _NBUF < _NCHUNKS:
        stores[j].wait()
        gathers[j + _NBUF] = gather(j + _NBUF)
    for j in range(max(0, _NCHUNKS - _NBUF), _NCHUNKS):
      stores[j].wait()

  return k


_sc_gather = _make_sc_gather()


def kernel(deriv_types, embedding_weight):
  idx = deriv_types.astype(jnp.int32).reshape(_NW, _NCHUNKS, _CHUNK)
  return _sc_gather(idx, embedding_weight)
